# initial kernel scaffold (unmeasured)
import jax
import jax.numpy as jnp
from jax import lax
from jax.experimental import pallas as pl
from jax.experimental.pallas import tpu as pltpu

N_DEV = 8
N_LAYERS = 3
N_SLOTS = 8


def kernel(x, Win0, Wout0, Win1, Wout1, Win2, Wout2):
    b, d = x.shape

    def body(x_ref, win0, wout0, win1, wout1, win2, wout2, out_ref,
             comm_ref, send_sems, recv_sems):
        my = lax.axis_index("i")
        right = lax.rem(my + 1, N_DEV)

        wins = [win0, win1, win2]
        wouts = [wout0, wout1, wout2]

        def contrib(act, l):
            h = jnp.maximum(
                jnp.dot(act, wins[l][...], preferred_element_type=jnp.float32),
                0.0,
            )
            return jnp.dot(h, wouts[l][...], preferred_element_type=jnp.float32)

        comm_ref[0, 0] = x_ref[...]
        comm_ref[0, 1] = contrib(x_ref[...], 0)

        hop = 0
        cur = 0
        for l in range(N_LAYERS):
            if l > 0:
                act = comm_ref[cur, 1]
                comm_ref[cur, 0] = act
                comm_ref[cur, 1] = contrib(act, l)
            for _ in range(N_DEV - 1):
                nxt = (hop + 1) % N_SLOTS
                rdma = pltpu.make_async_remote_copy(
                    src_ref=comm_ref.at[cur],
                    dst_ref=comm_ref.at[nxt],
                    send_sem=send_sems.at[nxt],
                    recv_sem=recv_sems.at[nxt],
                    device_id=(right,),
                    device_id_type=pl.DeviceIdType.MESH,
                )
                rdma.start()
                rdma.wait()
                hop += 1
                cur = nxt
                comm_ref[cur, 1] += contrib(comm_ref[cur, 0], l)

        my_slice = lax.rem(my + N_LAYERS, N_DEV)
        out_ref[pl.ds(my_slice * b, b), :] = comm_ref[cur, 1]

        for s in range(N_DEV - 1):
            nxt = (hop + 1) % N_SLOTS
            rdma = pltpu.make_async_remote_copy(
                src_ref=comm_ref.at[cur, 1],
                dst_ref=comm_ref.at[nxt, 1],
                send_sem=send_sems.at[nxt],
                recv_sem=recv_sems.at[nxt],
                device_id=(right,),
                device_id_type=pl.DeviceIdType.MESH,
            )
            rdma.start()
            rdma.wait()
            hop += 1
            cur = nxt
            sl = lax.rem(my + N_LAYERS - 1 - s + N_DEV, N_DEV)
            out_ref[pl.ds(sl * b, b), :] = comm_ref[cur, 1]

    return pl.pallas_call(
        body,
        out_shape=jax.ShapeDtypeStruct((N_DEV * b, d), jnp.float32),
        in_specs=[pl.BlockSpec(memory_space=pltpu.VMEM)] * 7,
        out_specs=pl.BlockSpec(memory_space=pltpu.VMEM),
        scratch_shapes=[
            pltpu.VMEM((N_SLOTS, 2, b, d), jnp.float32),
            pltpu.SemaphoreType.DMA((N_SLOTS,)),
            pltpu.SemaphoreType.DMA((N_SLOTS,)),
        ],
    )(x, Win0, Wout0, Win1, Wout1, Win2, Wout2)


# baseline (device time: 257187 ns/iter reference)
import jax
import jax.numpy as jnp
from jax import lax
from jax.experimental import pallas as pl
from jax.experimental.pallas import tpu as pltpu

N_DEV = 8
N_LAYERS = 3
N_SLOTS = 8


def kernel(x, Win0, Wout0, Win1, Wout1, Win2, Wout2):
    b, d = x.shape

    def body(x_ref, win0, wout0, win1, wout1, win2, wout2, out_ref,
             comm_ref, w_in, w_out, send_sems, recv_sems, w_sems):
        my = lax.axis_index("i")
        right = lax.rem(my + 1, N_DEV)

        wins = [win0, win1, win2]
        wouts = [wout0, wout1, wout2]

        def load_weights(l):
            cin = pltpu.make_async_copy(wins[l], w_in, w_sems.at[0])
            cout = pltpu.make_async_copy(wouts[l], w_out, w_sems.at[1])
            cin.start()
            cout.start()
            cin.wait()
            cout.wait()

        def contrib(act):
            h = jnp.maximum(
                jnp.dot(act, w_in[...], preferred_element_type=jnp.float32),
                0.0,
            )
            return jnp.dot(h, w_out[...], preferred_element_type=jnp.float32)

        load_weights(0)
        comm_ref[0, 0] = x_ref[...]
        comm_ref[0, 1] = contrib(x_ref[...])

        hop = 0
        cur = 0
        for l in range(N_LAYERS):
            if l > 0:
                load_weights(l)
                act = comm_ref[cur, 1]
                comm_ref[cur, 0] = act
                comm_ref[cur, 1] = contrib(act)
            for _ in range(N_DEV - 1):
                nxt = (hop + 1) % N_SLOTS
                rdma = pltpu.make_async_remote_copy(
                    src_ref=comm_ref.at[cur],
                    dst_ref=comm_ref.at[nxt],
                    send_sem=send_sems.at[nxt],
                    recv_sem=recv_sems.at[nxt],
                    device_id=(right,),
                    device_id_type=pl.DeviceIdType.MESH,
                )
                rdma.start()
                rdma.wait()
                hop += 1
                cur = nxt
                comm_ref[cur, 1] += contrib(comm_ref[cur, 0])

        my_slice = lax.rem(my + N_LAYERS, N_DEV)
        out_ref[pl.ds(my_slice * b, b), :] = comm_ref[cur, 1]

        for s in range(N_DEV - 1):
            nxt = (hop + 1) % N_SLOTS
            rdma = pltpu.make_async_remote_copy(
                src_ref=comm_ref.at[cur, 1],
                dst_ref=comm_ref.at[nxt, 1],
                send_sem=send_sems.at[nxt],
                recv_sem=recv_sems.at[nxt],
                device_id=(right,),
                device_id_type=pl.DeviceIdType.MESH,
            )
            rdma.start()
            rdma.wait()
            hop += 1
            cur = nxt
            sl = lax.rem(my + N_LAYERS - 1 - s + N_DEV, N_DEV)
            out_ref[pl.ds(sl * b, b), :] = comm_ref[cur, 1]

    return pl.pallas_call(
        body,
        out_shape=jax.ShapeDtypeStruct((N_DEV * b, d), jnp.float32),
        in_specs=[pl.BlockSpec(memory_space=pltpu.VMEM)]
        + [pl.BlockSpec(memory_space=pltpu.MemorySpace.HBM)] * 6,
        out_specs=pl.BlockSpec(memory_space=pltpu.VMEM),
        scratch_shapes=[
            pltpu.VMEM((N_SLOTS, 2, b, d), jnp.float32),
            pltpu.VMEM((d, Win0.shape[1]), jnp.float32),
            pltpu.VMEM((Win0.shape[1], d), jnp.float32),
            pltpu.SemaphoreType.DMA((N_SLOTS,)),
            pltpu.SemaphoreType.DMA((N_SLOTS,)),
            pltpu.SemaphoreType.DMA((2,)),
        ],
    )(x, Win0, Wout0, Win1, Wout1, Win2, Wout2)


# device time: 125649 ns/iter; 2.0469x vs baseline; 2.0469x over previous
import jax
import jax.numpy as jnp
from jax import lax
from jax.experimental import pallas as pl
from jax.experimental.pallas import tpu as pltpu

N_DEV = 8
N_LAYERS = 3
N_SLOTS = 8


def kernel(x, Win0, Wout0, Win1, Wout1, Win2, Wout2):
    b, d = x.shape

    def body(x_ref, win0, wout0, win1, wout1, win2, wout2, out_ref,
             act_ref, acc_ref, w_in, w_out,
             act_ssem, act_rsem, acc_ssem, acc_rsem, w_sems):
        my = lax.axis_index("i")
        right = lax.rem(my + 1, N_DEV)

        wins = [win0, win1, win2]
        wouts = [wout0, wout1, wout2]

        def load_weights(l):
            cin = pltpu.make_async_copy(wins[l], w_in, w_sems.at[0])
            cout = pltpu.make_async_copy(wouts[l], w_out, w_sems.at[1])
            cin.start()
            cout.start()
            cin.wait()
            cout.wait()

        def contrib(act):
            h = jnp.maximum(
                jnp.dot(act, w_in[...], preferred_element_type=jnp.float32),
                0.0,
            )
            return jnp.dot(h, w_out[...], preferred_element_type=jnp.float32)

        pend_act = {}
        pend_acc = {}

        def start(ring_ref, ssem, rsem, pend, s):
            ns = (s + 1) % N_SLOTS
            if s in pend:
                pend.pop(s).wait_send()
            r = pltpu.make_async_remote_copy(
                src_ref=ring_ref.at[s],
                dst_ref=ring_ref.at[ns],
                send_sem=ssem.at[s],
                recv_sem=rsem.at[ns],
                device_id=(right,),
                device_id_type=pl.DeviceIdType.MESH,
            )
            r.start()
            pend[s] = r
            return r

        def start_act(s):
            return start(act_ref, act_ssem, act_rsem, pend_act, s)

        def start_acc(s):
            return start(acc_ref, acc_ssem, acc_rsem, pend_acc, s)

        load_weights(0)
        act_ref[0] = x_ref[...].astype(jnp.bfloat16)

        cur = 0
        for l in range(N_LAYERS):
            if l > 0:
                load_weights(l)
                act_ref[cur] = acc_ref[cur]
            a = start_act(cur)
            delta = contrib(act_ref[cur])
            acc_ref[cur] = delta.astype(jnp.bfloat16)
            c = start_acc(cur)
            for h in range(N_DEV - 1):
                nxt = (cur + 1) % N_SLOTS
                a.wait_recv()
                if h < N_DEV - 2:
                    a = start_act(nxt)
                delta = contrib(act_ref[nxt])
                c.wait_recv()
                acc_ref[nxt] = (
                    acc_ref[nxt].astype(jnp.float32) + delta
                ).astype(jnp.bfloat16)
                if h < N_DEV - 2:
                    c = start_acc(nxt)
                cur = nxt

        my_slice = lax.rem(my + N_LAYERS, N_DEV)
        out_ref[pl.ds(my_slice * b, b), :] = acc_ref[cur].astype(jnp.float32)

        c = start_acc(cur)
        for s in range(N_DEV - 1):
            nxt = (cur + 1) % N_SLOTS
            c.wait_recv()
            if s < N_DEV - 2:
                c = start_acc(nxt)
            sl = lax.rem(my + N_LAYERS - 1 - s + N_DEV, N_DEV)
            out_ref[pl.ds(sl * b, b), :] = acc_ref[nxt].astype(jnp.float32)
            cur = nxt

        for r in pend_act.values():
            r.wait_send()
        for r in pend_acc.values():
            r.wait_send()

    return pl.pallas_call(
        body,
        out_shape=jax.ShapeDtypeStruct((N_DEV * b, d), jnp.float32),
        in_specs=[pl.BlockSpec(memory_space=pltpu.VMEM)]
        + [pl.BlockSpec(memory_space=pltpu.MemorySpace.HBM)] * 6,
        out_specs=pl.BlockSpec(memory_space=pltpu.VMEM),
        scratch_shapes=[
            pltpu.VMEM((N_SLOTS, b, d), jnp.bfloat16),
            pltpu.VMEM((N_SLOTS, b, d), jnp.bfloat16),
            pltpu.VMEM((d, Win0.shape[1]), jnp.float32),
            pltpu.VMEM((Win0.shape[1], d), jnp.float32),
            pltpu.SemaphoreType.DMA((N_SLOTS,)),
            pltpu.SemaphoreType.DMA((N_SLOTS,)),
            pltpu.SemaphoreType.DMA((N_SLOTS,)),
            pltpu.SemaphoreType.DMA((N_SLOTS,)),
            pltpu.SemaphoreType.DMA((2,)),
        ],
    )(x, Win0, Wout0, Win1, Wout1, Win2, Wout2)


# device time: 115010 ns/iter; 2.2362x vs baseline; 1.0925x over previous
import jax
import jax.numpy as jnp
from jax import lax
from jax.experimental import pallas as pl
from jax.experimental.pallas import tpu as pltpu

N_DEV = 8
N_LAYERS = 3
N_SLOTS = 8


def kernel(x, Win0, Wout0, Win1, Wout1, Win2, Wout2):
    b, d = x.shape
    hb = b // 2

    def body(x_ref, win0, wout0, win1, wout1, win2, wout2, out_ref,
             act_ref, acc_ref, w_in, w_out,
             tact_ss, tact_rs, bact_ss, bact_rs,
             tacc_ss, tacc_rs, bacc_ss, bacc_rs, w_sems):
        my = lax.axis_index("i")
        right = lax.rem(my + 1, N_DEV)
        left = lax.rem(my + N_DEV - 1, N_DEV)

        wins = [win0, win1, win2]
        wouts = [wout0, wout1, wout2]

        def load_weights(l):
            cin = pltpu.make_async_copy(wins[l], w_in, w_sems.at[0])
            cout = pltpu.make_async_copy(wouts[l], w_out, w_sems.at[1])
            cin.start()
            cout.start()
            cin.wait()
            cout.wait()

        def contrib(act):
            h = jnp.maximum(
                jnp.dot(act, w_in[...], preferred_element_type=jnp.float32),
                0.0,
            )
            return jnp.dot(h, w_out[...], preferred_element_type=jnp.float32)

        pend = [dict() for _ in range(4)]

        def start(ring_ref, row_off, dev, ssem, rsem, pend_d, s):
            ns = (s + 1) % N_SLOTS
            if s in pend_d:
                pend_d.pop(s).wait_send()
            r = pltpu.make_async_remote_copy(
                src_ref=ring_ref.at[s, pl.ds(row_off, hb)],
                dst_ref=ring_ref.at[ns, pl.ds(row_off, hb)],
                send_sem=ssem.at[s],
                recv_sem=rsem.at[ns],
                device_id=(dev,),
                device_id_type=pl.DeviceIdType.MESH,
            )
            r.start()
            pend_d[s] = r
            return r

        def start_act(s):
            return (
                start(act_ref, 0, right, tact_ss, tact_rs, pend[0], s),
                start(act_ref, hb, left, bact_ss, bact_rs, pend[1], s),
            )

        def start_acc(s):
            return (
                start(acc_ref, 0, right, tacc_ss, tacc_rs, pend[2], s),
                start(acc_ref, hb, left, bacc_ss, bacc_rs, pend[3], s),
            )

        load_weights(0)
        act_ref[0] = x_ref[...].astype(jnp.bfloat16)

        cur = 0
        for l in range(N_LAYERS):
            if l > 0:
                load_weights(l)
                act_ref[cur] = acc_ref[cur]
            a = start_act(cur)
            delta = contrib(act_ref[cur])
            acc_ref[cur] = delta.astype(jnp.bfloat16)
            c = start_acc(cur)
            for h in range(N_DEV - 1):
                nxt = (cur + 1) % N_SLOTS
                a[0].wait_recv()
                a[1].wait_recv()
                if h < N_DEV - 2:
                    a = start_act(nxt)
                delta = contrib(act_ref[nxt])
                c[0].wait_recv()
                c[1].wait_recv()
                acc_ref[nxt] = (
                    acc_ref[nxt].astype(jnp.float32) + delta
                ).astype(jnp.bfloat16)
                if h < N_DEV - 2:
                    c = start_acc(nxt)
                cur = nxt

        t_sl = lax.rem(my + N_LAYERS, N_DEV)
        b_sl = lax.rem(my + N_DEV - N_LAYERS, N_DEV)
        out_ref[pl.ds(t_sl * b, hb), :] = acc_ref[cur, :hb].astype(jnp.float32)
        out_ref[pl.ds(b_sl * b + hb, hb), :] = (
            acc_ref[cur, hb:].astype(jnp.float32)
        )

        c = start_acc(cur)
        for s in range(N_DEV - 1):
            nxt = (cur + 1) % N_SLOTS
            c[0].wait_recv()
            c[1].wait_recv()
            if s < N_DEV - 2:
                c = start_acc(nxt)
            t_sl = lax.rem(my + N_LAYERS - 1 - s + N_DEV, N_DEV)
            b_sl = lax.rem(my + s + 1 - N_LAYERS + N_DEV, N_DEV)
            out_ref[pl.ds(t_sl * b, hb), :] = (
                acc_ref[nxt, :hb].astype(jnp.float32)
            )
            out_ref[pl.ds(b_sl * b + hb, hb), :] = (
                acc_ref[nxt, hb:].astype(jnp.float32)
            )
            cur = nxt

        for pend_d in pend:
            for r in pend_d.values():
                r.wait_send()

    return pl.pallas_call(
        body,
        out_shape=jax.ShapeDtypeStruct((N_DEV * b, d), jnp.float32),
        in_specs=[pl.BlockSpec(memory_space=pltpu.VMEM)]
        + [pl.BlockSpec(memory_space=pltpu.MemorySpace.HBM)] * 6,
        out_specs=pl.BlockSpec(memory_space=pltpu.VMEM),
        scratch_shapes=[
            pltpu.VMEM((N_SLOTS, b, d), jnp.bfloat16),
            pltpu.VMEM((N_SLOTS, b, d), jnp.bfloat16),
            pltpu.VMEM((d, Win0.shape[1]), jnp.float32),
            pltpu.VMEM((Win0.shape[1], d), jnp.float32),
            pltpu.SemaphoreType.DMA((N_SLOTS,)),
            pltpu.SemaphoreType.DMA((N_SLOTS,)),
            pltpu.SemaphoreType.DMA((N_SLOTS,)),
            pltpu.SemaphoreType.DMA((N_SLOTS,)),
            pltpu.SemaphoreType.DMA((N_SLOTS,)),
            pltpu.SemaphoreType.DMA((N_SLOTS,)),
            pltpu.SemaphoreType.DMA((N_SLOTS,)),
            pltpu.SemaphoreType.DMA((N_SLOTS,)),
            pltpu.SemaphoreType.DMA((2,)),
        ],
    )(x, Win0, Wout0, Win1, Wout1, Win2, Wout2)
